# Initial kernel scaffold; baseline (speedup 1.0000x reference)
import jax
import jax.numpy as jnp
from jax.experimental import pallas as pl

INPUT_SIZE = 128
OUTPUT_SIZE = 100000
NUM_POS = 96
NUM_NOISE = 64
SFD = 8
N_CLASSES = 12
MOMENTUM = 0.5
B = 64

BM = 4000
N_NOISE_ROWS = NUM_NOISE * B
NOISE_END = NUM_POS + N_NOISE_ROWS


def _body(xpos_ref, xn_ref, vis_ref, lab_ref, mem_ref,
          sim_ref, nsim_ref, lwo_ref, newmem_ref):
    i = pl.program_id(0)
    f32 = jnp.float32

    xpos = xpos_ref[...]
    mem = mem_ref[...]

    sim_ref[...] = jax.lax.dot_general(
        xpos, mem, (((1,), (1,)), ((), ())), preferred_element_type=f32)

    def norm(upd):
        ss = jnp.sum(upd * upd, axis=1, keepdims=True)
        nrm = jnp.maximum(jnp.sqrt(ss), 1e-12)
        return upd / nrm

    @pl.when(i == 0)
    def _():
        lab = lab_ref[...]
        cls = jax.lax.broadcasted_iota(jnp.int32, (B, N_CLASSES), 1)
        eq = (lab == cls).astype(f32)
        cnt = jnp.sum(eq, axis=0, keepdims=True)
        denom = jnp.where(cnt == 0.0, 1.0, cnt)
        lwo = eq / denom
        lwo_ref[...] = lwo

        r0 = jax.lax.broadcasted_iota(jnp.int32, (NUM_POS, N_CLASSES), 0)
        r1 = jax.lax.broadcasted_iota(jnp.int32, (NUM_POS, N_CLASSES), 1)
        rrow = ((r0 // SFD) == r1).astype(f32)
        p1 = jax.lax.dot_general(rrow, lwo, (((1,), (1,)), ((), ())),
                                 preferred_element_type=f32)
        c0 = jax.lax.broadcasted_iota(jnp.int32, (B * SFD, B), 0)
        c1 = jax.lax.broadcasted_iota(jnp.int32, (B * SFD, B), 1)
        rcol = ((c0 // SFD) == c1).astype(f32)
        p2 = jax.lax.dot_general(p1, rcol, (((1,), (1,)), ((), ())),
                                 preferred_element_type=f32)
        m0 = jax.lax.broadcasted_iota(jnp.int32, (NUM_POS, B * SFD), 0)
        m1 = jax.lax.broadcasted_iota(jnp.int32, (NUM_POS, B * SFD), 1)
        pmat = p2 * ((m0 % SFD) == (m1 % SFD)).astype(f32)
        present = jnp.sum(pmat, axis=1, keepdims=True) > 0.5

        xv = xpos * vis_ref[...]
        get96 = jax.lax.dot_general(pmat, xv, (((1,), (0,)), ((), ())),
                                    preferred_element_type=f32)
        mem96 = mem[0:NUM_POS, :]
        pos_upd = MOMENTUM * mem96 + (1.0 - MOMENTUM) * jnp.where(
            present, get96, mem96)

        xn = xn_ref[...]
        nsim_ref[...] = jax.lax.dot_general(
            xn, mem96, (((1,), (1,)), ((), ())), preferred_element_type=f32)

        upd = jnp.concatenate([pos_upd, xn[0:BM - NUM_POS, :]], axis=0)
        newmem_ref[...] = norm(upd)

    @pl.when(i == 1)
    def _():
        xn = xn_ref[...]
        upd = jnp.concatenate(
            [xn[BM - NUM_POS:N_NOISE_ROWS, :], mem[NOISE_END - BM:, :]],
            axis=0)
        newmem_ref[...] = norm(upd)

    @pl.when(i >= 2)
    def _():
        newmem_ref[...] = norm(mem)


def kernel(x, y, visible, img_label, memory):
    xpos = x[:, :SFD, :].reshape(B * SFD, INPUT_SIZE)
    xn = x[:, SFD:, :].reshape(B * NUM_NOISE, INPUT_SIZE)
    vis = visible.reshape(B * SFD, 1)
    lab = img_label.astype(jnp.int32).reshape(B, 1)

    grid = (OUTPUT_SIZE // BM,)
    sim, nsim, lwo, new_memory = pl.pallas_call(
        _body,
        grid=grid,
        in_specs=[
            pl.BlockSpec((B * SFD, INPUT_SIZE), lambda i: (0, 0)),
            pl.BlockSpec((B * NUM_NOISE, INPUT_SIZE), lambda i: (0, 0)),
            pl.BlockSpec((B * SFD, 1), lambda i: (0, 0)),
            pl.BlockSpec((B, 1), lambda i: (0, 0)),
            pl.BlockSpec((BM, INPUT_SIZE), lambda i: (i, 0)),
        ],
        out_specs=[
            pl.BlockSpec((B * SFD, BM), lambda i: (0, i)),
            pl.BlockSpec((B * NUM_NOISE, NUM_POS), lambda i: (0, 0)),
            pl.BlockSpec((B, N_CLASSES), lambda i: (0, 0)),
            pl.BlockSpec((BM, INPUT_SIZE), lambda i: (i, 0)),
        ],
        out_shape=[
            jax.ShapeDtypeStruct((B * SFD, OUTPUT_SIZE), jnp.float32),
            jax.ShapeDtypeStruct((B * NUM_NOISE, NUM_POS), jnp.float32),
            jax.ShapeDtypeStruct((B, N_CLASSES), jnp.float32),
            jax.ShapeDtypeStruct((OUTPUT_SIZE, INPUT_SIZE), jnp.float32),
        ],
    )(xpos, xn, vis, lab, memory)

    similarity = sim.reshape(B, SFD, OUTPUT_SIZE)
    noise_similarity = nsim.reshape(B, NUM_NOISE, NUM_POS)
    y_idx = y.astype(jnp.int32)
    return (similarity, y_idx, noise_similarity, lwo, new_memory)


# speedup vs baseline: 2.3065x; 2.3065x over previous
import jax
import jax.numpy as jnp
from jax.experimental import pallas as pl

INPUT_SIZE = 128
OUTPUT_SIZE = 100000
NUM_POS = 96
NUM_NOISE = 64
SFD = 8
N_CLASSES = 12
MOMENTUM = 0.5
B = 64

BM = 4096
N_NOISE_ROWS = NUM_NOISE * B
NOISE_END = NUM_POS + N_NOISE_ROWS


def _body(xpos_ref, xn_ref, vis_ref, lab_ref, mem_ref,
          sim_ref, nsim_ref, lwo_ref, newmem_ref):
    i = pl.program_id(0)
    f32 = jnp.float32

    xpos = xpos_ref[...]
    mem = mem_ref[...]

    sim_ref[...] = jax.lax.dot_general(
        xpos, mem, (((1,), (1,)), ((), ())), preferred_element_type=f32)

    def norm(upd):
        ss = jnp.sum(upd * upd, axis=1, keepdims=True)
        nrm = jnp.maximum(jnp.sqrt(ss), 1e-12)
        return upd / nrm

    @pl.when(i == 0)
    def _():
        lab = lab_ref[...]
        cls = jax.lax.broadcasted_iota(jnp.int32, (B, N_CLASSES), 1)
        eq = (lab == cls).astype(f32)
        cnt = jnp.sum(eq, axis=0, keepdims=True)
        denom = jnp.where(cnt == 0.0, 1.0, cnt)
        lwo = eq / denom
        lwo_ref[...] = lwo

        r0 = jax.lax.broadcasted_iota(jnp.int32, (NUM_POS, N_CLASSES), 0)
        r1 = jax.lax.broadcasted_iota(jnp.int32, (NUM_POS, N_CLASSES), 1)
        rrow = ((r0 // SFD) == r1).astype(f32)
        p1 = jax.lax.dot_general(rrow, lwo, (((1,), (1,)), ((), ())),
                                 preferred_element_type=f32)
        c0 = jax.lax.broadcasted_iota(jnp.int32, (B * SFD, B), 0)
        c1 = jax.lax.broadcasted_iota(jnp.int32, (B * SFD, B), 1)
        rcol = ((c0 // SFD) == c1).astype(f32)
        p2 = jax.lax.dot_general(p1, rcol, (((1,), (1,)), ((), ())),
                                 preferred_element_type=f32)
        m0 = jax.lax.broadcasted_iota(jnp.int32, (NUM_POS, B * SFD), 0)
        m1 = jax.lax.broadcasted_iota(jnp.int32, (NUM_POS, B * SFD), 1)
        pmat = p2 * ((m0 % SFD) == (m1 % SFD)).astype(f32)
        present = jnp.sum(pmat, axis=1, keepdims=True) > 0.5

        xv = xpos * vis_ref[...]
        get96 = jax.lax.dot_general(pmat, xv, (((1,), (0,)), ((), ())),
                                    preferred_element_type=f32)
        mem96 = mem[0:NUM_POS, :]
        pos_upd = MOMENTUM * mem96 + (1.0 - MOMENTUM) * jnp.where(
            present, get96, mem96)

        xn = xn_ref[...]
        nsim_ref[...] = jax.lax.dot_general(
            xn, mem96, (((1,), (1,)), ((), ())), preferred_element_type=f32)

        upd = jnp.concatenate([pos_upd, xn[0:BM - NUM_POS, :]], axis=0)
        newmem_ref[...] = norm(upd)

    @pl.when(i == 1)
    def _():
        xn = xn_ref[...]
        upd = jnp.concatenate(
            [xn[BM - NUM_POS:N_NOISE_ROWS, :], mem[NOISE_END - BM:, :]],
            axis=0)
        newmem_ref[...] = norm(upd)

    @pl.when(i >= 2)
    def _():
        newmem_ref[...] = norm(mem)


def kernel(x, y, visible, img_label, memory):
    xpos = x[:, :SFD, :].reshape(B * SFD, INPUT_SIZE)
    xn = x[:, SFD:, :].reshape(B * NUM_NOISE, INPUT_SIZE)
    vis = visible.reshape(B * SFD, 1)
    lab = img_label.astype(jnp.int32).reshape(B, 1)

    grid = ((OUTPUT_SIZE + BM - 1) // BM,)
    sim, nsim, lwo, new_memory = pl.pallas_call(
        _body,
        grid=grid,
        in_specs=[
            pl.BlockSpec((B * SFD, INPUT_SIZE), lambda i: (0, 0)),
            pl.BlockSpec((B * NUM_NOISE, INPUT_SIZE), lambda i: (0, 0)),
            pl.BlockSpec((B * SFD, 1), lambda i: (0, 0)),
            pl.BlockSpec((B, 1), lambda i: (0, 0)),
            pl.BlockSpec((BM, INPUT_SIZE), lambda i: (i, 0)),
        ],
        out_specs=[
            pl.BlockSpec((B * SFD, BM), lambda i: (0, i)),
            pl.BlockSpec((B * NUM_NOISE, NUM_POS), lambda i: (0, 0)),
            pl.BlockSpec((B, N_CLASSES), lambda i: (0, 0)),
            pl.BlockSpec((BM, INPUT_SIZE), lambda i: (i, 0)),
        ],
        out_shape=[
            jax.ShapeDtypeStruct((B * SFD, OUTPUT_SIZE), jnp.float32),
            jax.ShapeDtypeStruct((B * NUM_NOISE, NUM_POS), jnp.float32),
            jax.ShapeDtypeStruct((B, N_CLASSES), jnp.float32),
            jax.ShapeDtypeStruct((OUTPUT_SIZE, INPUT_SIZE), jnp.float32),
        ],
    )(xpos, xn, vis, lab, memory)

    similarity = sim.reshape(B, SFD, OUTPUT_SIZE)
    noise_similarity = nsim.reshape(B, NUM_NOISE, NUM_POS)
    y_idx = y.astype(jnp.int32)
    return (similarity, y_idx, noise_similarity, lwo, new_memory)
